# baseline (speedup 1.0000x reference)
import jax
import jax.numpy as jnp
import numpy as np
from jax.experimental import pallas as pl
from jax.experimental.pallas import tpu as pltpu

_B, _C, _N = 4, 64, 1024
_U = 10


def _build_selection_matrix() -> np.ndarray:
    with jax.default_device(jax.local_devices(backend="cpu")[0]):
        skey = jax.random.key(42)
        idx = np.asarray(jax.random.randint(skey, (_C, _U), 0, _C))
    e = (idx.T.reshape(_U * _C, 1) == np.arange(_C)[None, :])
    return np.ascontiguousarray(e.astype(np.float32))


_E_SEL = _build_selection_matrix()


def _block_kernel(x_hbm, w1t_ref, b1_ref, w2t_ref, b2_ref, gamma_ref, e_ref,
                  out_hbm, x_ref, o_ref, in_sems, out_sems):
    rb = range(x_ref.shape[0])
    for b in rb:
        pltpu.make_async_copy(x_hbm.at[b], x_ref.at[b], in_sems.at[b]).start()
    xs = []
    for b in rb:
        pltpu.make_async_copy(x_hbm.at[b], x_ref.at[b], in_sems.at[b]).wait()
        xs.append(x_ref[b])

    qns = [jnp.sqrt(jnp.sum(xb * xb, axis=0, keepdims=True)) for xb in xs]
    qinvs = [jnp.where(qn > 0.0, 1.0 / qn, 0.0) for qn in qns]
    qhats = [xb * qi for xb, qi in zip(xs, qinvs)]
    sims = [jax.lax.dot_general(qh, qh, (((0,), (0,)), ((), ())),
                                preferred_element_type=jnp.float32)
            for qh in qhats]
    adjs = [(s > 0.5).astype(jnp.float32) for s in sims]
    rinvs = [1.0 / (jnp.sum(a, axis=0, keepdims=True) + 1.0) for a in adjs]

    qrs = [1.0 / jnp.sum(xb, axis=0, keepdims=True) for xb in xs]
    qrs = [jnp.where(jnp.isinf(r), 0.0, r) for r in qrs]
    qfs = [xb * r for xb, r in zip(xs, qrs)]

    w1t, w2t = w1t_ref[...].T, w2t_ref[...].T
    b1c, b2c = b1_ref[...], b2_ref[...]
    h1s = [jnp.dot(w1t, qf, preferred_element_type=jnp.float32) for qf in qfs]
    p1s = [jnp.dot(h1, a, preferred_element_type=jnp.float32) + h1
           for h1, a in zip(h1s, adjs)]
    hs = [jax.nn.relu(p1 * ri + b1c) for p1, ri in zip(p1s, rinvs)]
    h2s = [jnp.dot(w2t, h, preferred_element_type=jnp.float32) for h in hs]
    p2s = [jnp.dot(h2, a, preferred_element_type=jnp.float32) + h2
           for h2, a in zip(h2s, adjs)]
    qrys = [p2 * ri + b2c for p2, ri in zip(p2s, rinvs)]

    ec = e_ref[...]
    gs = [jnp.dot(ec, q, preferred_element_type=jnp.float32) for q in qrys]
    qks = [jnp.sum(g.reshape(_U, _C, _N) * q[None, :, :], axis=1)
           for g, q in zip(gs, qrys)]
    ms = [jnp.max(qk, axis=0, keepdims=True)
          - jnp.sum(qk, axis=0, keepdims=True) * (1.0 / 64.0) for qk in qks]
    m_all = jnp.concatenate(ms, axis=0)

    iota_n = jax.lax.broadcasted_iota(jnp.int32, (len(xs), _N), 1)
    onehot_rows = []
    m_work = m_all
    for _ in range(_U):
        mv = jnp.max(m_work, axis=1, keepdims=True)
        cand = jnp.where(m_work == mv, iota_n, _N)
        sel = jnp.min(cand, axis=1, keepdims=True)
        row = (iota_n == sel)
        onehot_rows.append(row.astype(jnp.float32))
        m_work = jnp.where(row, -jnp.inf, m_work)
    os_ = [jnp.concatenate([r[b:b + 1] for r in onehot_rows], axis=0)
           for b in rb]

    qreds = [jax.lax.dot_general(q, o, (((1,), (1,)), ((), ())),
                                 preferred_element_type=jnp.float32)
             for q, o in zip(qrys, os_)]
    scale = 1.0 / np.sqrt(float(_N))
    scs = [jax.lax.dot_general(qr, q, (((0,), (0,)), ((), ())),
                               preferred_element_type=jnp.float32) * scale
           for qr, q in zip(qreds, qrys)]
    exs = [jnp.exp(s - jnp.max(s, axis=1, keepdims=True)) for s in scs]
    attns = [e / jnp.sum(e, axis=1, keepdims=True) for e in exs]
    upds = [jax.lax.dot_general(q, a, (((1,), (1,)), ((), ())),
                                preferred_element_type=jnp.float32)
            for q, a in zip(qrys, attns)]

    ctxs = list(qrys)
    shift = 1
    while shift < _N:
        z = jnp.zeros((_C, shift), jnp.float32)
        ctxs = [c + jnp.concatenate([z, c[:, :-shift]], axis=1) for c in ctxs]
        shift *= 2
    masks = [jnp.sum(o, axis=0, keepdims=True) for o in os_]
    scats = [jnp.dot(u, o, preferred_element_type=jnp.float32)
             for u, o in zip(upds, os_)]
    gm = gamma_ref[0, 0]
    for b in rb:
        ctx = jnp.where(masks[b] > 0.0, scats[b], ctxs[b])
        o_ref[b] = gm * ctx + xs[b]
        pltpu.make_async_copy(o_ref.at[b], out_hbm.at[b],
                              out_sems.at[b]).start()
    for b in rb:
        pltpu.make_async_copy(o_ref.at[b], out_hbm.at[b],
                              out_sems.at[b]).wait()


def kernel(x, W1, b1, W2, b2, gamma):
    out = pl.pallas_call(
        _block_kernel,
        in_specs=[
            pl.BlockSpec(memory_space=pl.ANY),
            pl.BlockSpec(memory_space=pltpu.MemorySpace.VMEM),
            pl.BlockSpec(memory_space=pltpu.MemorySpace.VMEM),
            pl.BlockSpec(memory_space=pltpu.MemorySpace.VMEM),
            pl.BlockSpec(memory_space=pltpu.MemorySpace.VMEM),
            pl.BlockSpec(memory_space=pltpu.MemorySpace.VMEM),
            pl.BlockSpec(memory_space=pltpu.MemorySpace.VMEM),
        ],
        out_specs=pl.BlockSpec(memory_space=pl.ANY),
        out_shape=jax.ShapeDtypeStruct((_B, _C, _N), jnp.float32),
        scratch_shapes=[
            pltpu.VMEM((_B, _C, _N), jnp.float32),
            pltpu.VMEM((_B, _C, _N), jnp.float32),
            pltpu.SemaphoreType.DMA((_B,)),
            pltpu.SemaphoreType.DMA((_B,)),
        ],
    )(x, W1, b1.reshape(8, 1), W2, b2.reshape(_C, 1),
      gamma.reshape(1, 1), jnp.asarray(_E_SEL))
    return out


# speedup vs baseline: 1.1057x; 1.1057x over previous
import jax
import jax.numpy as jnp
import numpy as np
from jax.experimental import pallas as pl

_B, _C, _N = 4, 64, 1024
_U = 10


_INDEX_SAMPLE = np.array([[4, 18, 55, 1, 13, 43, 1, 39, 6, 2], [40, 50, 25, 27, 12, 18, 11, 2, 3, 7], [54, 11, 12, 3, 44, 17, 48, 27, 28, 55], [5, 36, 21, 46, 51, 20, 46, 50, 17, 45], [7, 4, 23, 61, 57, 0, 60, 36, 35, 13], [20, 27, 18, 51, 56, 55, 11, 18, 27, 57], [25, 6, 32, 8, 3, 57, 52, 32, 2, 57], [44, 5, 51, 45, 36, 60, 46, 42, 49, 33], [23, 16, 53, 44, 49, 56, 24, 56, 40, 62], [31, 21, 62, 56, 19, 25, 55, 31, 58, 33], [49, 28, 37, 36, 63, 12, 62, 34, 25, 25], [59, 63, 35, 60, 1, 35, 5, 8, 30, 35], [3, 0, 2, 3, 34, 20, 14, 6, 17, 28], [23, 34, 34, 29, 47, 38, 25, 42, 17, 1], [7, 12, 27, 28, 18, 38, 43, 3, 49, 33], [7, 50, 43, 48, 32, 19, 46, 17, 11, 26], [46, 20, 22, 19, 14, 27, 15, 31, 24, 47], [39, 52, 36, 33, 22, 15, 46, 8, 34, 51], [4, 37, 54, 7, 63, 6, 5, 56, 44, 21], [45, 45, 52, 13, 23, 19, 0, 11, 54, 62], [41, 41, 49, 37, 31, 48, 2, 34, 47, 33], [41, 15, 25, 52, 23, 51, 61, 50, 11, 57], [4, 12, 49, 43, 48, 45, 32, 20, 28, 52], [61, 9, 31, 25, 54, 43, 40, 20, 55, 37], [53, 0, 32, 58, 17, 57, 21, 24, 0, 42], [34, 33, 60, 39, 58, 16, 26, 13, 0, 47], [36, 59, 15, 59, 0, 21, 62, 26, 10, 24], [23, 2, 56, 62, 7, 8, 1, 28, 58, 37], [45, 45, 51, 32, 22, 3, 3, 49, 26, 53], [39, 11, 36, 49, 13, 27, 27, 16, 15, 23], [55, 14, 62, 12, 2, 31, 7, 32, 27, 19], [43, 40, 60, 16, 40, 17, 36, 13, 15, 10], [17, 7, 48, 61, 62, 62, 36, 8, 8, 11], [10, 36, 2, 44, 12, 44, 33, 63, 54, 11], [52, 17, 57, 21, 14, 24, 51, 26, 30, 17], [39, 52, 46, 43, 20, 18, 60, 47, 2, 60], [58, 44, 36, 30, 41, 44, 0, 6, 1, 46], [36, 59, 48, 37, 22, 44, 34, 62, 55, 57], [0, 4, 33, 7, 8, 47, 56, 10, 11, 59], [59, 16, 29, 55, 35, 56, 50, 8, 44, 28], [37, 34, 10, 17, 29, 22, 31, 34, 27, 13], [2, 46, 29, 48, 59, 50, 17, 10, 6, 57], [32, 5, 27, 63, 5, 31, 55, 7, 53, 21], [52, 33, 44, 28, 37, 50, 0, 23, 33, 22], [12, 55, 52, 49, 52, 53, 43, 31, 7, 32], [48, 30, 29, 44, 31, 26, 27, 41, 48, 26], [3, 56, 43, 44, 55, 23, 58, 10, 60, 20], [18, 36, 62, 11, 35, 6, 25, 60, 8, 0], [19, 24, 1, 16, 18, 54, 55, 56, 26, 60], [10, 32, 20, 20, 36, 48, 17, 31, 62, 8], [12, 41, 18, 56, 11, 9, 18, 25, 53, 40], [58, 62, 22, 52, 40, 51, 0, 5, 41, 3], [36, 12, 3, 42, 26, 61, 31, 46, 20, 29], [60, 15, 34, 43, 9, 11, 62, 21, 13, 49], [44, 19, 53, 21, 62, 50, 4, 45, 5, 6], [5, 28, 0, 56, 43, 57, 46, 27, 21, 46], [57, 26, 27, 32, 8, 14, 14, 1, 17, 15], [17, 6, 46, 12, 25, 11, 5, 49, 33, 37], [16, 27, 19, 22, 10, 33, 28, 11, 24, 25], [37, 13, 38, 3, 19, 36, 58, 51, 5, 28], [11, 47, 4, 33, 42, 30, 48, 36, 9, 44], [6, 29, 0, 15, 32, 46, 44, 27, 57, 10], [53, 43, 62, 54, 47, 63, 41, 60, 6, 9], [7, 9, 21, 2, 18, 51, 30, 16, 18, 7]], dtype=np.int32)


def _build_selection_matrix() -> np.ndarray:
    e = (_INDEX_SAMPLE.T.reshape(_U * _C, 1) == np.arange(_C)[None, :])
    return np.ascontiguousarray(e.astype(np.float32))


_E_SEL = _build_selection_matrix()


def _block_kernel(x_ref, w1_ref, b1_ref, w2_ref, b2_ref, gamma_ref, e_ref,
                  out_ref):
    rb = range(x_ref.shape[0])
    xs = [x_ref[b] for b in rb]

    qns = [jnp.sqrt(jnp.sum(xb * xb, axis=0, keepdims=True)) for xb in xs]
    qinvs = [jnp.where(qn > 0.0, 1.0 / qn, 0.0) for qn in qns]
    qhats = [xb * qi for xb, qi in zip(xs, qinvs)]
    sims = [jax.lax.dot_general(qh, qh, (((0,), (0,)), ((), ())),
                                preferred_element_type=jnp.float32)
            for qh in qhats]
    adjs = [(s > 0.5).astype(jnp.float32) for s in sims]

    qrs = [1.0 / jnp.sum(xb, axis=0, keepdims=True) for xb in xs]
    qrs = [jnp.where(jnp.isinf(r), 0.0, r) for r in qrs]
    qfs = [xb * r for xb, r in zip(xs, qrs)]

    w1t, w2t = w1_ref[...].T, w2_ref[...].T
    b1c, b2c = b1_ref[...], b2_ref[...]
    h1s = [jnp.dot(w1t, qf, preferred_element_type=jnp.float32) for qf in qfs]
    ones_row = jnp.ones((1, _N), jnp.float32)
    h1es = [jnp.concatenate([h1, ones_row], axis=0) for h1 in h1s]
    p1es = [jnp.dot(h1e, a, preferred_element_type=jnp.float32) + h1e
            for h1e, a in zip(h1es, adjs)]
    rinvs = [1.0 / p1e[8:9] for p1e in p1es]
    hs = [jax.nn.relu(p1e[:8] * ri + b1c) for p1e, ri in zip(p1es, rinvs)]
    p2s = [jnp.dot(h, a, preferred_element_type=jnp.float32) + h
           for h, a in zip(hs, adjs)]
    qrys = [jnp.dot(w2t, p2, preferred_element_type=jnp.float32) * ri + b2c
            for p2, ri in zip(p2s, rinvs)]

    ctxs = list(qrys)
    shift = 1
    while shift < _N:
        z = jnp.zeros((_C, shift), jnp.float32)
        ctxs = [c + jnp.concatenate([z, c[:, :-shift]], axis=1) for c in ctxs]
        shift *= 2

    ec = e_ref[...]
    gs = [jnp.dot(ec, q, preferred_element_type=jnp.float32) for q in qrys]
    qks = [jnp.sum(g.reshape(_U, _C, _N) * q[None, :, :], axis=1)
           for g, q in zip(gs, qrys)]
    ms = [jnp.max(qk, axis=0, keepdims=True)
          - jnp.sum(qk, axis=0, keepdims=True) * (1.0 / 64.0) for qk in qks]
    m_all = jnp.concatenate(ms, axis=0)

    iota_n = jax.lax.broadcasted_iota(jnp.int32, (len(xs), _N), 1)
    onehot_rows = []
    m_work = m_all
    for _ in range(_U):
        mv = jnp.max(m_work, axis=1, keepdims=True)
        cand = jnp.where(m_work == mv, iota_n, _N)
        sel = jnp.min(cand, axis=1, keepdims=True)
        row = (iota_n == sel)
        onehot_rows.append(row.astype(jnp.float32))
        m_work = jnp.where(row, -jnp.inf, m_work)
    os_ = [jnp.concatenate([r[b:b + 1] for r in onehot_rows], axis=0)
           for b in rb]

    qreds = [jax.lax.dot_general(q, o, (((1,), (1,)), ((), ())),
                                 preferred_element_type=jnp.float32)
             for q, o in zip(qrys, os_)]
    scale = 1.0 / np.sqrt(float(_N))
    scs = [jax.lax.dot_general(qr, q, (((0,), (0,)), ((), ())),
                               preferred_element_type=jnp.float32) * scale
           for qr, q in zip(qreds, qrys)]
    exs = [jnp.exp(s - jnp.max(s, axis=1, keepdims=True)) for s in scs]
    attns = [e / jnp.sum(e, axis=1, keepdims=True) for e in exs]
    upds = [jax.lax.dot_general(q, a, (((1,), (1,)), ((), ())),
                                preferred_element_type=jnp.float32)
            for q, a in zip(qrys, attns)]

    masks = [jnp.sum(o, axis=0, keepdims=True) for o in os_]
    scats = [jnp.dot(u, o, preferred_element_type=jnp.float32)
             for u, o in zip(upds, os_)]
    gm = gamma_ref[0, 0]
    for b in rb:
        ctx = jnp.where(masks[b] > 0.0, scats[b], ctxs[b])
        out_ref[b] = gm * ctx + xs[b]


def kernel(x, W1, b1, W2, b2, gamma):
    out = pl.pallas_call(
        _block_kernel,
        out_shape=jax.ShapeDtypeStruct((_B, _C, _N), jnp.float32),
    )(x, W1, b1.reshape(8, 1), W2, b2.reshape(_C, 1),
      gamma.reshape(1, 1), jnp.asarray(_E_SEL))
    return out
